# Initial kernel scaffold; baseline (speedup 1.0000x reference)
import functools
import jax
import jax.numpy as jnp
from jax import lax
from jax.experimental import pallas as pl
from jax.experimental.pallas import tpu as pltpu
from jax.experimental.pallas import tpu_sc as plsc

USER = 10000
ITEM = 10000
D = 128
E = 320000
NC = 2
NS = 16
CH = 128
NCHUNK = (E // NS + CH - 1) // CH
EPW = NCHUNK * CH
ROWS_PER_SUB = USER // NS


def _sc_spmm_body(emb_hbm, src_hbm, dst_hbm, vals_hbm, zeros_hbm, out_hbm,
                  src_v, dst_v, vals_v, rows_v, acc, sem):
    c = lax.axis_index("c")
    s = lax.axis_index("s")
    wid = c * NS + s

    pltpu.sync_copy(zeros_hbm, acc.at[pl.ds(s * ROWS_PER_SUB, ROWS_PER_SUB)])
    pltpu.sync_copy(src_hbm.at[wid], src_v)
    pltpu.sync_copy(dst_hbm.at[wid], dst_v)
    pltpu.sync_copy(vals_hbm.at[wid], vals_v)
    plsc.subcore_barrier()

    def chunk_body(j, carry):
        idx = src_v.at[pl.ds(j * CH, CH)]
        pltpu.async_copy(emb_hbm.at[idx], rows_v, sem).wait()

        def edge_body(e, carry2):
            v = vals_v[j * CH + e]
            for k in range(D // 16):
                sl = pl.ds(k * 16, 16)
                rows_v[e, sl] = rows_v[e, sl] * v
            return carry2

        lax.fori_loop(0, CH, edge_body, 0)
        pltpu.sync_copy(rows_v, acc.at[dst_v.at[j]], add=True)
        return carry

    lax.fori_loop(0, NCHUNK, chunk_body, 0)
    plsc.subcore_barrier()

    pltpu.sync_copy(acc.at[pl.ds(s * ROWS_PER_SUB, ROWS_PER_SUB)],
                    out_hbm.at[pl.ds(c * USER + s * ROWS_PER_SUB, ROWS_PER_SUB)])


@jax.jit
def _sc_spmm(emb_cat, src_all, dst_all, vals_all, zeros):
    mesh = plsc.VectorSubcoreMesh(core_axis_name="c", subcore_axis_name="s")
    f = pl.kernel(
        _sc_spmm_body,
        out_type=jax.ShapeDtypeStruct((USER + ITEM, D), jnp.float32),
        mesh=mesh,
        scratch_types=[
            pltpu.VMEM((EPW,), jnp.int32),
            pltpu.VMEM((NCHUNK, CH), jnp.int32),
            pltpu.VMEM((EPW,), jnp.float32),
            pltpu.VMEM((CH, D), jnp.float32),
            pltpu.VMEM_SHARED((USER, D), jnp.float32),
            pltpu.SemaphoreType.DMA,
        ],
    )
    return f(emb_cat, src_all, dst_all, vals_all, zeros)


def _tc_dense_body(s_ref, e_ref, w_ref, b_ref, o_ref):
    x = 2.0 * (jnp.maximum(s_ref[...], 0.0) + e_ref[...])
    y = jnp.dot(x, w_ref[...], preferred_element_type=jnp.float32) + b_ref[...]
    o_ref[...] = jnp.maximum(y, 0.0)


@jax.jit
def _tc_dense(S, emb_cat, W, b2):
    n = USER + ITEM
    blk = 2000
    return pl.pallas_call(
        _tc_dense_body,
        grid=(n // blk,),
        in_specs=[
            pl.BlockSpec((blk, D), lambda i: (i, 0)),
            pl.BlockSpec((blk, D), lambda i: (i, 0)),
            pl.BlockSpec((D, D), lambda i: (0, 0)),
            pl.BlockSpec((1, D), lambda i: (0, 0)),
        ],
        out_specs=pl.BlockSpec((blk, D), lambda i: (i, 0)),
        out_shape=jax.ShapeDtypeStruct((n, D), jnp.float32),
    )(S, emb_cat, W, b2)


def _prep_edges(edge_index, edge_vals, src_offset):
    src = edge_index[1].astype(jnp.int32) + src_offset
    dst = edge_index[0].astype(jnp.int32)
    vals = edge_vals.astype(jnp.float32)
    pad = NS * EPW - E
    src = jnp.concatenate([src, jnp.zeros((pad,), jnp.int32)])
    dst = jnp.concatenate([dst, jnp.zeros((pad,), jnp.int32)])
    vals = jnp.concatenate([vals, jnp.zeros((pad,), jnp.float32)])
    return src.reshape(NS, EPW), dst.reshape(NS, NCHUNK, CH), vals.reshape(NS, EPW)


def kernel(uu_edge_index, uu_edge_vals, ii_edge_index, ii_edge_vals,
           uEmbeds, iEmbeds, W, b):
    su, du, vu = _prep_edges(uu_edge_index, uu_edge_vals, 0)
    si, di, vi = _prep_edges(ii_edge_index, ii_edge_vals, USER)
    src_all = jnp.concatenate([su, si], axis=0)
    dst_all = jnp.concatenate([du, di], axis=0)
    vals_all = jnp.concatenate([vu, vi], axis=0)
    emb_cat = jnp.concatenate([uEmbeds, iEmbeds], axis=0)
    zeros = jnp.zeros((ROWS_PER_SUB, D), jnp.float32)

    S = _sc_spmm(emb_cat, src_all, dst_all, vals_all, zeros)
    return _tc_dense(S, emb_cat, W, b.reshape(1, D))


# speedup vs baseline: 3.2485x; 3.2485x over previous
import functools
import jax
import jax.numpy as jnp
from jax import lax
from jax.experimental import pallas as pl
from jax.experimental.pallas import tpu as pltpu
from jax.experimental.pallas import tpu_sc as plsc

USER = 10000
ITEM = 10000
D = 128
E = 320000
NC = 2
NS = 16
CH = 128
CPB = 8
NBLK = 20
EPB = CPB * CH
EPW = NBLK * EPB
ROWS_PER_SUB = 640
LAST_ROWS = USER - 15 * ROWS_PER_SUB


def _sc_spmm_body(emb_hbm, src_hbm, dst_hbm, vals_hbm, zeros_hbm, out_hbm,
                  src_b, dst_b, vals_b, rows_v, acc, sem):
    c = lax.axis_index("c")
    s = lax.axis_index("s")
    wid = c * NS + s

    @pl.when(s < NS - 1)
    def _():
        pltpu.sync_copy(zeros_hbm, acc.at[pl.ds(s * ROWS_PER_SUB, ROWS_PER_SUB)])

    @pl.when(s == NS - 1)
    def _():
        pltpu.sync_copy(zeros_hbm.at[pl.ds(0, LAST_ROWS)],
                        acc.at[pl.ds((NS - 1) * ROWS_PER_SUB, LAST_ROWS)])
    plsc.subcore_barrier()

    def blk_body(bi, carry):
        pltpu.sync_copy(src_hbm.at[wid, bi], src_b)
        pltpu.sync_copy(dst_hbm.at[wid, bi], dst_b)
        pltpu.sync_copy(vals_hbm.at[wid, bi], vals_b)

        def chunk_body(j, carry1):
            idx = src_b.at[pl.ds(j * CH, CH)]
            pltpu.async_copy(emb_hbm.at[idx], rows_v, sem).wait()

            def grp_body(g, carry2):
                vv = vals_b[pl.ds(j * CH + g * 16, 16)]
                for t in range(16):
                    e = g * 16 + t
                    v = vv[t]
                    for k in range(D // 16):
                        sl = pl.ds(k * 16, 16)
                        rows_v[e, sl] = rows_v[e, sl] * v
                return carry2

            lax.fori_loop(0, CH // 16, grp_body, 0)
            pltpu.sync_copy(rows_v, acc.at[dst_b.at[j]], add=True)
            return carry1

        lax.fori_loop(0, CPB, chunk_body, 0)
        return carry

    lax.fori_loop(0, NBLK, blk_body, 0)
    plsc.subcore_barrier()

    @pl.when(s < NS - 1)
    def _():
        pltpu.sync_copy(acc.at[pl.ds(s * ROWS_PER_SUB, ROWS_PER_SUB)],
                        out_hbm.at[pl.ds(c * USER + s * ROWS_PER_SUB, ROWS_PER_SUB)])

    @pl.when(s == NS - 1)
    def _():
        pltpu.sync_copy(acc.at[pl.ds((NS - 1) * ROWS_PER_SUB, LAST_ROWS)],
                        out_hbm.at[pl.ds(c * USER + (NS - 1) * ROWS_PER_SUB, LAST_ROWS)])


@jax.jit
def _sc_spmm(emb_cat, src_all, dst_all, vals_all, zeros):
    mesh = plsc.VectorSubcoreMesh(core_axis_name="c", subcore_axis_name="s")
    f = pl.kernel(
        _sc_spmm_body,
        out_type=jax.ShapeDtypeStruct((USER + ITEM, D), jnp.float32),
        mesh=mesh,
        scratch_types=[
            pltpu.VMEM((EPB,), jnp.int32),
            pltpu.VMEM((CPB, CH), jnp.int32),
            pltpu.VMEM((EPB,), jnp.float32),
            pltpu.VMEM((CH, D), jnp.float32),
            pltpu.VMEM_SHARED((USER, D), jnp.float32),
            pltpu.SemaphoreType.DMA,
        ],
    )
    return f(emb_cat, src_all, dst_all, vals_all, zeros)


def _tc_dense_body(s_ref, e_ref, w_ref, b_ref, o_ref):
    x = 2.0 * (jnp.maximum(s_ref[...], 0.0) + e_ref[...])
    y = jnp.dot(x, w_ref[...], preferred_element_type=jnp.float32) + b_ref[...]
    o_ref[...] = jnp.maximum(y, 0.0)


@jax.jit
def _tc_dense(S, emb_cat, W, b2):
    n = USER + ITEM
    blk = 2000
    return pl.pallas_call(
        _tc_dense_body,
        grid=(n // blk,),
        in_specs=[
            pl.BlockSpec((blk, D), lambda i: (i, 0)),
            pl.BlockSpec((blk, D), lambda i: (i, 0)),
            pl.BlockSpec((D, D), lambda i: (0, 0)),
            pl.BlockSpec((1, D), lambda i: (0, 0)),
        ],
        out_specs=pl.BlockSpec((blk, D), lambda i: (i, 0)),
        out_shape=jax.ShapeDtypeStruct((n, D), jnp.float32),
    )(S, emb_cat, W, b2)


def _prep_edges(edge_index, edge_vals, src_offset):
    src = edge_index[1].astype(jnp.int32) + src_offset
    dst = edge_index[0].astype(jnp.int32)
    vals = edge_vals.astype(jnp.float32)
    pad = NS * EPW - E
    src = jnp.concatenate([src, jnp.zeros((pad,), jnp.int32)])
    dst = jnp.concatenate([dst, jnp.zeros((pad,), jnp.int32)])
    vals = jnp.concatenate([vals, jnp.zeros((pad,), jnp.float32)])
    return (src.reshape(NS, NBLK, EPB),
            dst.reshape(NS, NBLK, CPB, CH),
            vals.reshape(NS, NBLK, EPB))


def kernel(uu_edge_index, uu_edge_vals, ii_edge_index, ii_edge_vals,
           uEmbeds, iEmbeds, W, b):
    su, du, vu = _prep_edges(uu_edge_index, uu_edge_vals, 0)
    si, di, vi = _prep_edges(ii_edge_index, ii_edge_vals, USER)
    src_all = jnp.concatenate([su, si], axis=0)
    dst_all = jnp.concatenate([du, di], axis=0)
    vals_all = jnp.concatenate([vu, vi], axis=0)
    emb_cat = jnp.concatenate([uEmbeds, iEmbeds], axis=0)
    zeros = jnp.zeros((ROWS_PER_SUB, D), jnp.float32)

    S = _sc_spmm(emb_cat, src_all, dst_all, vals_all, zeros)
    return _tc_dense(S, emb_cat, W, b.reshape(1, D))
